# baseline (speedup 1.0000x reference)
import functools

import jax
import jax.numpy as jnp
from jax import lax
from jax.experimental import pallas as pl
from jax.experimental.pallas import tpu as pltpu
from jax.experimental.pallas import tpu_sc as plsc

N_NEWS, N_KEY, N_STOCK = 50000, 20000, 5000
NP_NEWS, NP_KEY, NP_STOCK = 50176, 20224, 5120
E_HK, E_HS, E_L = 500000, 250000, 100000
EP_HK, EP_HS, EP_L = 503808, 258048, 114688
NSUB = 16
NB = 3
WSEG = 256
KCT = 6
KDE = 4
F32 = jnp.float32
I32 = jnp.int32

_MESH = plsc.VectorSubcoreMesh(core_axis_name="c", subcore_axis_name="s",
                               num_cores=2, num_subcores=NSUB)
_SC_PARAMS = pltpu.CompilerParams(use_tc_tiling_on_sc=False)

_RELS = ((EP_HK, NP_KEY, 0), (EP_HK, NP_NEWS, 1),
         (EP_HS, NP_STOCK, 0), (EP_HS, NP_NEWS, 2))
_CNT_CORE = (0, 1, 0, 1)


def _linear_body(x_ref, w_ref, b_ref, o_ref, *, act):
    acc = lax.dot_general(x_ref[...], w_ref[...], (((1,), (1,)), ((), ())),
                          preferred_element_type=F32)
    acc = acc + b_ref[...][None, :]
    if act:
        acc = jnp.maximum(acc, 0.0)
    o_ref[...] = acc


def tc_linear(x, W, b, act=False, blk=512):
    n, k = x.shape
    m = W.shape[0]
    return pl.pallas_call(
        functools.partial(_linear_body, act=act),
        grid=(pl.cdiv(n, blk),),
        in_specs=[
            pl.BlockSpec((blk, k), lambda i: (i, 0)),
            pl.BlockSpec((m, k), lambda i: (0, 0)),
            pl.BlockSpec((m,), lambda i: (0,)),
        ],
        out_specs=pl.BlockSpec((blk, m), lambda i: (i, 0)),
        out_shape=jax.ShapeDtypeStruct((n, m), F32),
    )(x, W, b)


def _sage_out_body(*refs, act, nrel, post):
    o_ref = refs[-1]
    x_ref, w_ref, b_ref = refs[3 * nrel], refs[3 * nrel + 1], refs[3 * nrel + 2]
    acc = lax.dot_general(x_ref[...], w_ref[...], (((1,), (1,)), ((), ())),
                          preferred_element_type=F32)
    acc = acc + b_ref[...][None, :]
    for r in range(nrel):
        ss4 = refs[3 * r][...]
        ic = refs[3 * r + 1][...]
        wl = refs[3 * r + 2][...]
        for q in range(4):
            m = ss4[q] * ic
            acc = acc + lax.dot_general(
                m, wl[:, q * 32:(q + 1) * 32], (((1,), (1,)), ((), ())),
                preferred_element_type=F32)
    if act:
        acc = jnp.maximum(acc, 0.0)
    if post:
        wp_ref, bp_ref = refs[3 * nrel + 3], refs[3 * nrel + 4]
        acc = lax.dot_general(acc, wp_ref[...], (((1,), (1,)), ((), ())),
                              preferred_element_type=F32)
        acc = acc + bp_ref[...][None, :]
    o_ref[...] = acc


def tc_sage_out(ss_ic_wl, x, W, b, act, post=None, blk=512):
    n, k = x.shape
    m = W.shape[0]
    nrel = len(ss_ic_wl)
    in_specs, args = [], []
    for ss4, ic, wl in ss_ic_wl:
        in_specs.append(pl.BlockSpec((4, blk, 32), lambda i: (0, i, 0)))
        in_specs.append(pl.BlockSpec((blk, 1), lambda i: (i, 0)))
        in_specs.append(pl.BlockSpec((m, m), lambda i: (0, 0)))
        args += [ss4, ic, wl]
    in_specs += [
        pl.BlockSpec((blk, k), lambda i: (i, 0)),
        pl.BlockSpec((m, k), lambda i: (0, 0)),
        pl.BlockSpec((m,), lambda i: (0,)),
    ]
    args += [x, W, b]
    mo = m
    if post is not None:
        Wp, bp = post
        mo = Wp.shape[0]
        in_specs += [
            pl.BlockSpec((mo, m), lambda i: (0, 0)),
            pl.BlockSpec((mo,), lambda i: (0,)),
        ]
        args += [Wp, bp]
    return pl.pallas_call(
        functools.partial(_sage_out_body, act=act, nrel=nrel,
                          post=post is not None),
        grid=(pl.cdiv(n, blk),),
        in_specs=in_specs,
        out_specs=pl.BlockSpec((blk, mo), lambda i: (i, 0)),
        out_shape=jax.ShapeDtypeStruct((n, mo), F32),
    )(*args)


def _decode2_body(a1_ref, b1_ref, a2_ref, b2r_ref, w_ref, bb_ref,
                  o1_ref, o2_ref):
    w = w_ref[...]
    s1 = jnp.maximum(a1_ref[...] + b1_ref[...], 0.0)
    o1_ref[...] = jnp.sum(s1 * w[0][None, :], axis=1) + bb_ref[0]
    s2 = jnp.maximum(a2_ref[...] + b2r_ref[...], 0.0)
    o2_ref[...] = jnp.sum(s2 * w[1][None, :], axis=1) + bb_ref[1]


def tc_decode2(a1, b1, a2, b2, w2, bb, blk=1024):
    n, m = a1.shape
    return pl.pallas_call(
        _decode2_body,
        grid=(pl.cdiv(n, blk),),
        in_specs=[
            pl.BlockSpec((blk, m), lambda i: (i, 0)),
            pl.BlockSpec((blk, m), lambda i: (i, 0)),
            pl.BlockSpec((blk, m), lambda i: (i, 0)),
            pl.BlockSpec((blk, m), lambda i: (i, 0)),
            pl.BlockSpec((2, m), lambda i: (0, 0)),
            pl.BlockSpec((2,), lambda i: (0,)),
        ],
        out_specs=[pl.BlockSpec((blk,), lambda i: (i,)),
                   pl.BlockSpec((blk,), lambda i: (i,))],
        out_shape=[jax.ShapeDtypeStruct((n,), F32),
                   jax.ShapeDtypeStruct((n,), F32)],
    )(a1, b1, a2, b2, w2, bb)


def _sc_segsum_layer(hn4, hk4, hs4, gd, zeros2d):
    out_type = [jax.ShapeDtypeStruct((4, npd, 32), F32) for _, npd, _ in _RELS]
    scratch = ([pltpu.VMEM_SHARED((NP_NEWS, 32), F32)]
               + [pltpu.VMEM((NB * WSEG,), I32) for _ in range(4)]
               + [pltpu.VMEM((WSEG, 32), F32) for _ in range(NB)]
               + [pltpu.SemaphoreType.DMA for _ in range(2 * NB + 5)])

    @functools.partial(pl.kernel, out_type=out_type, mesh=_MESH,
                       scratch_types=scratch, compiler_params=_SC_PARAMS)
    def k(t0, t1, t2, g0, d0, g1, d1, g2, d2, g3, d3, z2, o0, o1, o2, o3,
          acc, gib0, gib1, dib0, dib1, *rest):
        gidx2 = (gib0, gib1)
        dst2 = (dib0, dib1)
        rowb = rest[:NB]
        gsem = rest[NB:2 * NB]
        ssem = rest[2 * NB:3 * NB]
        pgsem = rest[3 * NB:3 * NB + 2]
        pdsem = rest[3 * NB + 2:3 * NB + 4]
        zsem = rest[3 * NB + 4]
        c = lax.axis_index("c")
        s = lax.axis_index("s")
        tables = (t0, t1, t2)
        gs = (g0, g1, g2, g3)
        ds = (d0, d1, d2, d3)
        outs = (o0, o1, o2, o3)
        for rel, (ep, npd, ti) in enumerate(_RELS):
            tab = tables[ti]
            gi, dd, out = gs[rel], ds[rel], outs[rel]
            per_s = ep // NSUB
            nsg = per_s // (NB * WSEG)
            stripe = npd // NSUB
            for qq in range(2):
                q = c * 2 + qq
                for j in range(NB):
                    pltpu.sync_copy(z2.at[pl.ds(0, WSEG)], rowb[j])
                nz = 0
                for off in range(0, stripe, WSEG):
                    sz = min(WSEG, stripe - off)
                    pltpu.async_copy(rowb[nz % NB].at[pl.ds(0, sz)],
                                     acc.at[pl.ds(s * stripe + off, sz)],
                                     zsem)
                    nz += 1
                for off in range(0, stripe, WSEG):
                    sz = min(WSEG, stripe - off)
                    pltpu.make_async_copy(
                        rowb[0].at[pl.ds(0, sz)],
                        acc.at[pl.ds(s * stripe + off, sz)], zsem).wait()
                base0 = s * per_s
                pltpu.async_copy(gi.at[q, pl.ds(base0, NB * WSEG)],
                                 gidx2[0], pgsem[0])
                pltpu.async_copy(dd.at[pl.ds(base0, NB * WSEG)],
                                 dst2[0], pdsem[0])
                plsc.subcore_barrier()

                def sg(g, carry, gi=gi, dd=dd, tab=tab, q=q, base0=base0,
                       nsg=nsg):
                    for buf in (0, 1):
                        @pl.when(g % 2 == buf)
                        def _(buf=buf):
                            gib, dib = gidx2[buf], dst2[buf]
                            gio, dio = gidx2[1 - buf], dst2[1 - buf]
                            b0 = base0 + g * (NB * WSEG)
                            pltpu.make_async_copy(
                                gi.at[q, pl.ds(b0, NB * WSEG)], gib,
                                pgsem[buf]).wait()
                            pltpu.make_async_copy(
                                dd.at[pl.ds(b0, NB * WSEG)], dib,
                                pdsem[buf]).wait()
                            for j in range(NB):
                                @pl.when(g > 0)
                                def _(j=j):
                                    pltpu.make_async_copy(
                                        rowb[j],
                                        acc.at[dio.at[pl.ds(j * WSEG, WSEG)]],
                                        ssem[j]).wait()
                                pltpu.async_copy(
                                    tab.at[gib.at[pl.ds(j * WSEG, WSEG)]],
                                    rowb[j], gsem[j])

                            @pl.when(g + 1 < nsg)
                            def _():
                                b1 = b0 + NB * WSEG
                                pltpu.async_copy(
                                    gi.at[q, pl.ds(b1, NB * WSEG)], gio,
                                    pgsem[1 - buf])
                                pltpu.async_copy(
                                    dd.at[pl.ds(b1, NB * WSEG)], dio,
                                    pdsem[1 - buf])
                            for j in range(NB):
                                pltpu.make_async_copy(
                                    tab.at[gib.at[pl.ds(j * WSEG, WSEG)]],
                                    rowb[j], gsem[j]).wait()
                                pltpu.async_copy(
                                    rowb[j],
                                    acc.at[dib.at[pl.ds(j * WSEG, WSEG)]],
                                    ssem[j], add=True)
                    return carry

                lax.fori_loop(0, nsg, sg, 0)
                dlast = dst2[(nsg - 1) % 2]
                for j in range(NB):
                    pltpu.make_async_copy(
                        rowb[j], acc.at[dlast.at[pl.ds(j * WSEG, WSEG)]],
                        ssem[j]).wait()
                plsc.subcore_barrier()
                pltpu.sync_copy(acc.at[pl.ds(s * stripe, stripe)],
                                out.at[q, pl.ds(s * stripe, stripe)])
                plsc.subcore_barrier()

    (g0, d0), (g1, d1), (g2, d2), (g3, d3) = gd
    return k(hn4, hk4, hs4, g0, d0, g1, d1, g2, d2, g3, d3, zeros2d)


def _sc_inv_counts(dsts, zeros1d, ones128):
    out_type = [jax.ShapeDtypeStruct((npd,), F32) for _, npd, _ in _RELS]
    scratch = ([pltpu.VMEM_SHARED((npd,), F32) for _, npd, _ in _RELS]
               + [pltpu.VMEM((4096,), F32), pltpu.VMEM((128,), F32),
                  pltpu.VMEM((NP_NEWS // NSUB,), F32)]
               + [pltpu.VMEM((KCT * 128,), I32) for _ in range(2)]
               + [pltpu.SemaphoreType.DMA for _ in range(KCT + 2)])

    @functools.partial(pl.kernel, out_type=out_type, mesh=_MESH,
                       scratch_types=scratch, compiler_params=_SC_PARAMS)
    def k(d0, d1, d2, d3, z1, ones_h, o0, o1, o2, o3,
          a0, a1, a2, a3, zb, onesb, cbuf, *rest):
        dst2 = rest[:2]
        ssem = rest[2:2 + KCT]
        pdsem = rest[2 + KCT:4 + KCT]
        c = lax.axis_index("c")
        s = lax.axis_index("s")
        dd = (d0, d1, d2, d3)
        outs = (o0, o1, o2, o3)
        accs = (a0, a1, a2, a3)
        pltpu.sync_copy(z1, zb)
        pltpu.sync_copy(ones_h, onesb)
        for rel, (ep, npd, _) in enumerate(_RELS):
            stripe = npd // NSUB

            @pl.when(c == _CNT_CORE[rel])
            def _(rel=rel, stripe=stripe):
                acc = accs[rel]
                pltpu.sync_copy(zb.at[pl.ds(0, stripe)],
                                acc.at[pl.ds(s * stripe, stripe)])

        plsc.subcore_barrier()
        for rel, (ep, npd, _) in enumerate(_RELS):
            per_s = ep // NSUB
            ngrp = per_s // (KCT * 128)

            @pl.when(c == _CNT_CORE[rel])
            def _(rel=rel, per_s=per_s, ngrp=ngrp):
                acc = accs[rel]
                base0 = s * per_s
                pltpu.async_copy(dd[rel].at[pl.ds(base0, KCT * 128)],
                                 dst2[0], pdsem[0])

                def grp(g, carry, rel=rel, base0=base0, ngrp=ngrp, acc=acc):
                    for buf in (0, 1):
                        @pl.when(g % 2 == buf)
                        def _(buf=buf):
                            dib = dst2[buf]
                            dio = dst2[1 - buf]
                            b0 = base0 + g * (KCT * 128)
                            pltpu.make_async_copy(
                                dd[rel].at[pl.ds(b0, KCT * 128)], dib,
                                pdsem[buf]).wait()
                            for j in range(KCT):
                                @pl.when(g > 0)
                                def _(j=j):
                                    pltpu.make_async_copy(
                                        onesb,
                                        acc.at[dio.at[pl.ds(j * 128, 128)]],
                                        ssem[j]).wait()

                            @pl.when(g + 1 < ngrp)
                            def _():
                                pltpu.async_copy(
                                    dd[rel].at[pl.ds(b0 + KCT * 128,
                                                     KCT * 128)],
                                    dio, pdsem[1 - buf])
                            for j in range(KCT):
                                pltpu.async_copy(
                                    onesb,
                                    acc.at[dib.at[pl.ds(j * 128, 128)]],
                                    ssem[j], add=True)
                    return carry

                lax.fori_loop(0, ngrp, grp, 0)
                dlast = dst2[(ngrp - 1) % 2]
                for j in range(KCT):
                    pltpu.make_async_copy(
                        onesb, acc.at[dlast.at[pl.ds(j * 128, 128)]],
                        ssem[j]).wait()

        plsc.subcore_barrier()
        for rel, (ep, npd, _) in enumerate(_RELS):
            stripe = npd // NSUB

            @pl.when(c == _CNT_CORE[rel])
            def _(rel=rel, stripe=stripe):
                acc = accs[rel]
                pltpu.sync_copy(acc.at[pl.ds(s * stripe, stripe)],
                                cbuf.at[pl.ds(0, stripe)])

                def rec(i, carry):
                    v = cbuf[pl.ds(i * 16, 16)]
                    cbuf[pl.ds(i * 16, 16)] = 1.0 / jnp.maximum(v, 1.0)
                    return carry

                lax.fori_loop(0, stripe // 16, rec, 0)
                pltpu.sync_copy(cbuf.at[pl.ds(0, stripe)],
                                outs[rel].at[pl.ds(s * stripe, stripe)])

    return k(dsts[0], dsts[1], dsts[2], dsts[3], zeros1d, ones128)


def _sc_decoder_gather(tables, idxs):
    out_type = [jax.ShapeDtypeStruct((EP_L, 128), F32) for _ in range(4)]
    scratch = ([pltpu.VMEM((KDE * 128,), I32) for _ in range(2)]
               + [pltpu.VMEM((128, 128), F32) for _ in range(KDE)]
               + [pltpu.SemaphoreType.DMA for _ in range(2 * KDE + 2)])

    @functools.partial(pl.kernel, out_type=out_type, mesh=_MESH,
                       scratch_types=scratch, compiler_params=_SC_PARAMS)
    def k(t0, t1, t2, t3, i0, i1, i2, i3, o0, o1, o2, o3, *rest):
        idx2 = rest[:2]
        rowb = rest[2:2 + KDE]
        gsem = rest[2 + KDE:2 + 2 * KDE]
        osem = rest[2 + 2 * KDE:2 + 3 * KDE]
        pisem = rest[2 + 3 * KDE:4 + 3 * KDE]
        c = lax.axis_index("c")
        s = lax.axis_index("s")
        w = s * 2 + c
        per_w = EP_L // 32
        ngrp = per_w // (KDE * 128)
        tabs = (t0, t1, t2, t3)
        iis = (i0, i1, i2, i3)
        outs = (o0, o1, o2, o3)
        for job in range(4):
            base0 = w * per_w
            pltpu.async_copy(iis[job].at[pl.ds(base0, KDE * 128)],
                             idx2[0], pisem[0])

            def grp(g, carry, job=job, base0=base0):
                for buf in (0, 1):
                    @pl.when(g % 2 == buf)
                    def _(buf=buf):
                        iib = idx2[buf]
                        iio = idx2[1 - buf]
                        b0 = base0 + g * (KDE * 128)
                        pltpu.make_async_copy(
                            iis[job].at[pl.ds(b0, KDE * 128)], iib,
                            pisem[buf]).wait()
                        for j in range(KDE):
                            @pl.when(g > 0)
                            def _(j=j):
                                pltpu.make_async_copy(
                                    rowb[j],
                                    outs[job].at[pl.ds(b0 - KDE * 128
                                                       + j * 128, 128)],
                                    osem[j]).wait()
                            pltpu.async_copy(
                                tabs[job].at[iib.at[pl.ds(j * 128, 128)]],
                                rowb[j], gsem[j])

                        @pl.when(g + 1 < ngrp)
                        def _():
                            pltpu.async_copy(
                                iis[job].at[pl.ds(b0 + KDE * 128, KDE * 128)],
                                iio, pisem[1 - buf])
                        for j in range(KDE):
                            pltpu.make_async_copy(
                                tabs[job].at[iib.at[pl.ds(j * 128, 128)]],
                                rowb[j], gsem[j]).wait()
                            pltpu.async_copy(
                                rowb[j],
                                outs[job].at[pl.ds(b0 + j * 128, 128)],
                                osem[j])
                return carry

            lax.fori_loop(0, ngrp, grp, 0)
            blast = base0 + (ngrp - 1) * (KDE * 128)
            for j in range(KDE):
                pltpu.make_async_copy(
                    rowb[j], outs[job].at[pl.ds(blast + j * 128, 128)],
                    osem[j]).wait()

    return k(tables[0], tables[1], tables[2], tables[3],
             idxs[0], idxs[1], idxs[2], idxs[3])


def _prep_edges(ei, n_src, n_dst, n_dst_pad, e_pad):
    src, dst = ei[0], ei[1]
    npad = e_pad - src.shape[0]
    ar = jnp.arange(npad, dtype=I32)
    srcp = jnp.concatenate([src, ar % jnp.int32(n_src)])
    dstp = jnp.concatenate([dst, n_dst + (ar % jnp.int32(n_dst_pad - n_dst))])
    gidx = srcp[None, :] * 4 + jnp.arange(4, dtype=I32)[:, None]
    return gidx, dstp


def _pad_idx(idx, n_rows, e_pad):
    npad = e_pad - idx.shape[0]
    ar = jnp.arange(npad, dtype=I32)
    return jnp.concatenate([idx, ar % jnp.int32(n_rows)])


def kernel(x_news, x_keyword, x_stock, ei_hk, ei_rhk, ei_hs, ei_rhs, eli_hk, eli_hs, news_W, news_b, key_W, key_b, stock_W, stock_b, c1_hk_Wl, c1_hk_bl, c1_hk_Wr, c1_rhk_Wl, c1_rhk_bl, c1_rhk_Wr, c1_hs_Wl, c1_hs_bl, c1_hs_Wr, c1_rhs_Wl, c1_rhs_bl, c1_rhs_Wr, c2_hk_Wl, c2_hk_bl, c2_hk_Wr, c2_rhk_Wl, c2_rhk_bl, c2_rhk_Wr, c2_hs_Wl, c2_hs_bl, c2_hs_Wr, c2_rhs_Wl, c2_rhs_bl, c2_rhs_Wr, dec_hk_W1, dec_hk_b1, dec_hk_W2, dec_hk_b2, dec_hs_W1, dec_hs_b1, dec_hs_W2, dec_hs_b2):
    h_news = tc_linear(x_news, news_W, news_b, act=True)
    h_key = tc_linear(x_keyword, key_W, key_b, act=True)
    h_stock = tc_linear(x_stock, stock_W, stock_b, act=True)

    zeros2d = jnp.zeros((512, 32), F32)
    zeros1d = jnp.zeros((4096,), F32)
    ones128 = jnp.ones((128,), F32)

    gd = [
        _prep_edges(ei_hk, N_NEWS, N_KEY, NP_KEY, EP_HK),
        _prep_edges(ei_rhk, N_KEY, N_NEWS, NP_NEWS, EP_HK),
        _prep_edges(ei_hs, N_NEWS, N_STOCK, NP_STOCK, EP_HS),
        _prep_edges(ei_rhs, N_STOCK, N_NEWS, NP_NEWS, EP_HS),
    ]
    ics = _sc_inv_counts([d for _, d in gd], zeros1d, ones128)
    ic_hk, ic_rhk, ic_hs, ic_rhs = [ic.reshape(-1, 1) for ic in ics]

    def layer(hn, hk, hs, Wls, bls, Wrs, act, posts=(None, None, None)):
        (wl_hk, wl_rhk, wl_hs, wl_rhs) = Wls
        (b_hk, b_rhk, b_hs, b_rhs) = bls
        (r_hk, r_rhk, r_hs, r_rhs) = Wrs
        ss = _sc_segsum_layer(hn.reshape(-1, 32), hk.reshape(-1, 32),
                              hs.reshape(-1, 32), gd, zeros2d)
        ss_hk, ss_rhk, ss_hs, ss_rhs = ss
        k = tc_sage_out([(ss_hk, ic_hk, wl_hk)], hk, r_hk, b_hk, act,
                        post=posts[1])
        n = tc_sage_out([(ss_rhk, ic_rhk, wl_rhk), (ss_rhs, ic_rhs, wl_rhs)],
                        hn, r_rhk + r_rhs, b_rhk + b_rhs, act, post=posts[0])
        s = tc_sage_out([(ss_hs, ic_hs, wl_hs)], hs, r_hs, b_hs, act,
                        post=posts[2])
        return n, k, s

    n1, k1, s1 = layer(
        h_news, h_key, h_stock,
        (c1_hk_Wl, c1_rhk_Wl, c1_hs_Wl, c1_rhs_Wl),
        (c1_hk_bl, c1_rhk_bl, c1_hs_bl, c1_rhs_bl),
        (c1_hk_Wr, c1_rhk_Wr, c1_hs_Wr, c1_rhs_Wr), True)
    post_n = (jnp.concatenate([dec_hk_W1[:, :128], dec_hs_W1[:, :128]], 0),
              jnp.zeros((256,), F32))
    post_k = (dec_hk_W1[:, 128:], dec_hk_b1)
    post_s = (dec_hs_W1[:, 128:], dec_hs_b1)
    za, b_hk, b_hs = layer(
        n1, k1, s1,
        (c2_hk_Wl, c2_rhk_Wl, c2_hs_Wl, c2_rhs_Wl),
        (c2_hk_bl, c2_rhk_bl, c2_hs_bl, c2_rhs_bl),
        (c2_hk_Wr, c2_rhk_Wr, c2_hs_Wr, c2_rhs_Wr), False,
        posts=(post_n, post_k, post_s))
    a2 = za.reshape(-1, 128)

    idxs = [2 * _pad_idx(eli_hk[0], N_NEWS, EP_L),
            _pad_idx(eli_hk[1], N_KEY, EP_L),
            2 * _pad_idx(eli_hs[0], N_NEWS, EP_L) + 1,
            _pad_idx(eli_hs[1], N_STOCK, EP_L)]
    ga_hk, gb_hk, ga_hs, gb_hs = _sc_decoder_gather(
        (a2, b_hk, a2, b_hs), idxs)

    pred_hk, pred_hs = tc_decode2(
        ga_hk, gb_hk, ga_hs, gb_hs,
        jnp.stack([dec_hk_W2[0], dec_hs_W2[0]]),
        jnp.concatenate([dec_hk_b2, dec_hs_b2]))
    return (pred_hk[:E_L], pred_hs[:E_L])


# speedup vs baseline: 1.0171x; 1.0171x over previous
import functools

import jax
import jax.numpy as jnp
from jax import lax
from jax.experimental import pallas as pl
from jax.experimental.pallas import tpu as pltpu
from jax.experimental.pallas import tpu_sc as plsc

N_NEWS, N_KEY, N_STOCK = 50000, 20000, 5000
NP_NEWS, NP_KEY, NP_STOCK = 50176, 20224, 5120
E_HK, E_HS, E_L = 500000, 250000, 100000
EP_HK, EP_HS, EP_L = 503808, 258048, 114688
NSUB = 16
NB = 6
KCT = 6
KDE = 7
F32 = jnp.float32
I32 = jnp.int32

_MESH = plsc.VectorSubcoreMesh(core_axis_name="c", subcore_axis_name="s",
                               num_cores=2, num_subcores=NSUB)
_SC_PARAMS = pltpu.CompilerParams(use_tc_tiling_on_sc=False)

_RELS = ((EP_HK, NP_KEY, 0), (EP_HK, NP_NEWS, 1),
         (EP_HS, NP_STOCK, 0), (EP_HS, NP_NEWS, 2))
_CNT_CORE = (0, 1, 0, 1)


def _linear_body(x_ref, w_ref, b_ref, o_ref, *, act):
    acc = lax.dot_general(x_ref[...], w_ref[...], (((1,), (1,)), ((), ())),
                          preferred_element_type=F32)
    acc = acc + b_ref[...][None, :]
    if act:
        acc = jnp.maximum(acc, 0.0)
    o_ref[...] = acc


def tc_linear(x, W, b, act=False, blk=512):
    n, k = x.shape
    m = W.shape[0]
    return pl.pallas_call(
        functools.partial(_linear_body, act=act),
        grid=(pl.cdiv(n, blk),),
        in_specs=[
            pl.BlockSpec((blk, k), lambda i: (i, 0)),
            pl.BlockSpec((m, k), lambda i: (0, 0)),
            pl.BlockSpec((m,), lambda i: (0,)),
        ],
        out_specs=pl.BlockSpec((blk, m), lambda i: (i, 0)),
        out_shape=jax.ShapeDtypeStruct((n, m), F32),
    )(x, W, b)


def _sage_out_body(*refs, act, nrel, post):
    o_ref = refs[-1]
    x_ref, w_ref, b_ref = refs[3 * nrel], refs[3 * nrel + 1], refs[3 * nrel + 2]
    acc = lax.dot_general(x_ref[...], w_ref[...], (((1,), (1,)), ((), ())),
                          preferred_element_type=F32)
    acc = acc + b_ref[...][None, :]
    for r in range(nrel):
        ss4 = refs[3 * r][...]
        ic = refs[3 * r + 1][...]
        wl = refs[3 * r + 2][...]
        for q in range(4):
            m = ss4[q] * ic
            acc = acc + lax.dot_general(
                m, wl[:, q * 32:(q + 1) * 32], (((1,), (1,)), ((), ())),
                preferred_element_type=F32)
    if act:
        acc = jnp.maximum(acc, 0.0)
    if post:
        wp_ref, bp_ref = refs[3 * nrel + 3], refs[3 * nrel + 4]
        acc = lax.dot_general(acc, wp_ref[...], (((1,), (1,)), ((), ())),
                              preferred_element_type=F32)
        acc = acc + bp_ref[...][None, :]
    o_ref[...] = acc


def tc_sage_out(ss_ic_wl, x, W, b, act, post=None, blk=512):
    n, k = x.shape
    m = W.shape[0]
    nrel = len(ss_ic_wl)
    in_specs, args = [], []
    for ss4, ic, wl in ss_ic_wl:
        in_specs.append(pl.BlockSpec((4, blk, 32), lambda i: (0, i, 0)))
        in_specs.append(pl.BlockSpec((blk, 1), lambda i: (i, 0)))
        in_specs.append(pl.BlockSpec((m, m), lambda i: (0, 0)))
        args += [ss4, ic, wl]
    in_specs += [
        pl.BlockSpec((blk, k), lambda i: (i, 0)),
        pl.BlockSpec((m, k), lambda i: (0, 0)),
        pl.BlockSpec((m,), lambda i: (0,)),
    ]
    args += [x, W, b]
    mo = m
    if post is not None:
        Wp, bp = post
        mo = Wp.shape[0]
        in_specs += [
            pl.BlockSpec((mo, m), lambda i: (0, 0)),
            pl.BlockSpec((mo,), lambda i: (0,)),
        ]
        args += [Wp, bp]
    return pl.pallas_call(
        functools.partial(_sage_out_body, act=act, nrel=nrel,
                          post=post is not None),
        grid=(pl.cdiv(n, blk),),
        in_specs=in_specs,
        out_specs=pl.BlockSpec((blk, mo), lambda i: (i, 0)),
        out_shape=jax.ShapeDtypeStruct((n, mo), F32),
    )(*args)


def _decode2_body(a1_ref, b1_ref, a2_ref, b2r_ref, w_ref, bb_ref,
                  o1_ref, o2_ref):
    w = w_ref[...]
    s1 = jnp.maximum(a1_ref[...] + b1_ref[...], 0.0)
    o1_ref[...] = jnp.sum(s1 * w[0][None, :], axis=1) + bb_ref[0]
    s2 = jnp.maximum(a2_ref[...] + b2r_ref[...], 0.0)
    o2_ref[...] = jnp.sum(s2 * w[1][None, :], axis=1) + bb_ref[1]


def tc_decode2(a1, b1, a2, b2, w2, bb, blk=1024):
    n, m = a1.shape
    return pl.pallas_call(
        _decode2_body,
        grid=(pl.cdiv(n, blk),),
        in_specs=[
            pl.BlockSpec((blk, m), lambda i: (i, 0)),
            pl.BlockSpec((blk, m), lambda i: (i, 0)),
            pl.BlockSpec((blk, m), lambda i: (i, 0)),
            pl.BlockSpec((blk, m), lambda i: (i, 0)),
            pl.BlockSpec((2, m), lambda i: (0, 0)),
            pl.BlockSpec((2,), lambda i: (0,)),
        ],
        out_specs=[pl.BlockSpec((blk,), lambda i: (i,)),
                   pl.BlockSpec((blk,), lambda i: (i,))],
        out_shape=[jax.ShapeDtypeStruct((n,), F32),
                   jax.ShapeDtypeStruct((n,), F32)],
    )(a1, b1, a2, b2, w2, bb)


def _sc_segsum_layer(hn4, hk4, hs4, gd, zeros2d):
    out_type = [jax.ShapeDtypeStruct((4, npd, 32), F32) for _, npd, _ in _RELS]
    scratch = ([pltpu.VMEM_SHARED((NP_NEWS, 32), F32)]
               + [pltpu.VMEM((NB * 128,), I32) for _ in range(4)]
               + [pltpu.VMEM((128, 32), F32) for _ in range(NB)]
               + [pltpu.SemaphoreType.DMA for _ in range(2 * NB + 5)])

    @functools.partial(pl.kernel, out_type=out_type, mesh=_MESH,
                       scratch_types=scratch, compiler_params=_SC_PARAMS)
    def k(t0, t1, t2, g0, d0, g1, d1, g2, d2, g3, d3, z2, o0, o1, o2, o3,
          acc, gib0, gib1, dib0, dib1, *rest):
        gidx2 = (gib0, gib1)
        dst2 = (dib0, dib1)
        rowb = rest[:NB]
        gsem = rest[NB:2 * NB]
        ssem = rest[2 * NB:3 * NB]
        pgsem = rest[3 * NB:3 * NB + 2]
        pdsem = rest[3 * NB + 2:3 * NB + 4]
        zsem = rest[3 * NB + 4]
        c = lax.axis_index("c")
        s = lax.axis_index("s")
        tables = (t0, t1, t2)
        gs = (g0, g1, g2, g3)
        ds = (d0, d1, d2, d3)
        outs = (o0, o1, o2, o3)
        for rel, (ep, npd, ti) in enumerate(_RELS):
            tab = tables[ti]
            gi, dd, out = gs[rel], ds[rel], outs[rel]
            per_s = ep // NSUB
            nsg = per_s // (NB * 128)
            stripe = npd // NSUB
            for qq in range(2):
                q = c * 2 + qq
                for j in range(NB):
                    pltpu.sync_copy(z2.at[pl.ds(0, 128)], rowb[j])
                nz = 0
                for off in range(0, stripe, 128):
                    sz = min(128, stripe - off)
                    pltpu.async_copy(rowb[nz % NB].at[pl.ds(0, sz)],
                                     acc.at[pl.ds(s * stripe + off, sz)],
                                     zsem)
                    nz += 1
                for off in range(0, stripe, 128):
                    sz = min(128, stripe - off)
                    pltpu.make_async_copy(
                        rowb[0].at[pl.ds(0, sz)],
                        acc.at[pl.ds(s * stripe + off, sz)], zsem).wait()
                base0 = s * per_s
                pltpu.async_copy(gi.at[q, pl.ds(base0, NB * 128)],
                                 gidx2[0], pgsem[0])
                pltpu.async_copy(dd.at[pl.ds(base0, NB * 128)],
                                 dst2[0], pdsem[0])
                plsc.subcore_barrier()

                def sg(g, carry, gi=gi, dd=dd, tab=tab, q=q, base0=base0,
                       nsg=nsg):
                    for buf in (0, 1):
                        @pl.when(g % 2 == buf)
                        def _(buf=buf):
                            gib, dib = gidx2[buf], dst2[buf]
                            gio, dio = gidx2[1 - buf], dst2[1 - buf]
                            b0 = base0 + g * (NB * 128)
                            pltpu.make_async_copy(
                                gi.at[q, pl.ds(b0, NB * 128)], gib,
                                pgsem[buf]).wait()
                            pltpu.make_async_copy(
                                dd.at[pl.ds(b0, NB * 128)], dib,
                                pdsem[buf]).wait()
                            for j in range(NB):
                                @pl.when(g > 0)
                                def _(j=j):
                                    pltpu.make_async_copy(
                                        rowb[j],
                                        acc.at[dio.at[pl.ds(j * 128, 128)]],
                                        ssem[j]).wait()
                                pltpu.async_copy(
                                    tab.at[gib.at[pl.ds(j * 128, 128)]],
                                    rowb[j], gsem[j])

                            @pl.when(g + 1 < nsg)
                            def _():
                                b1 = b0 + NB * 128
                                pltpu.async_copy(
                                    gi.at[q, pl.ds(b1, NB * 128)], gio,
                                    pgsem[1 - buf])
                                pltpu.async_copy(
                                    dd.at[pl.ds(b1, NB * 128)], dio,
                                    pdsem[1 - buf])
                            for j in range(NB):
                                pltpu.make_async_copy(
                                    tab.at[gib.at[pl.ds(j * 128, 128)]],
                                    rowb[j], gsem[j]).wait()
                                pltpu.async_copy(
                                    rowb[j],
                                    acc.at[dib.at[pl.ds(j * 128, 128)]],
                                    ssem[j], add=True)
                    return carry

                lax.fori_loop(0, nsg, sg, 0)
                dlast = dst2[(nsg - 1) % 2]
                for j in range(NB):
                    pltpu.make_async_copy(
                        rowb[j], acc.at[dlast.at[pl.ds(j * 128, 128)]],
                        ssem[j]).wait()
                plsc.subcore_barrier()
                pltpu.sync_copy(acc.at[pl.ds(s * stripe, stripe)],
                                out.at[q, pl.ds(s * stripe, stripe)])
                plsc.subcore_barrier()

    (g0, d0), (g1, d1), (g2, d2), (g3, d3) = gd
    return k(hn4, hk4, hs4, g0, d0, g1, d1, g2, d2, g3, d3, zeros2d)


def _sc_inv_counts(dsts, zeros1d, ones128):
    out_type = [jax.ShapeDtypeStruct((npd,), F32) for _, npd, _ in _RELS]
    scratch = ([pltpu.VMEM_SHARED((npd,), F32) for _, npd, _ in _RELS]
               + [pltpu.VMEM((4096,), F32), pltpu.VMEM((128,), F32),
                  pltpu.VMEM((NP_NEWS // NSUB,), F32)]
               + [pltpu.VMEM((KCT * 128,), I32) for _ in range(2)]
               + [pltpu.SemaphoreType.DMA for _ in range(KCT + 2)])

    @functools.partial(pl.kernel, out_type=out_type, mesh=_MESH,
                       scratch_types=scratch, compiler_params=_SC_PARAMS)
    def k(d0, d1, d2, d3, z1, ones_h, o0, o1, o2, o3,
          a0, a1, a2, a3, zb, onesb, cbuf, *rest):
        dst2 = rest[:2]
        ssem = rest[2:2 + KCT]
        pdsem = rest[2 + KCT:4 + KCT]
        c = lax.axis_index("c")
        s = lax.axis_index("s")
        dd = (d0, d1, d2, d3)
        outs = (o0, o1, o2, o3)
        accs = (a0, a1, a2, a3)
        pltpu.sync_copy(z1, zb)
        pltpu.sync_copy(ones_h, onesb)
        for rel, (ep, npd, _) in enumerate(_RELS):
            stripe = npd // NSUB

            @pl.when(c == _CNT_CORE[rel])
            def _(rel=rel, stripe=stripe):
                acc = accs[rel]
                pltpu.sync_copy(zb.at[pl.ds(0, stripe)],
                                acc.at[pl.ds(s * stripe, stripe)])

        plsc.subcore_barrier()
        for rel, (ep, npd, _) in enumerate(_RELS):
            per_s = ep // NSUB
            ngrp = per_s // (KCT * 128)

            @pl.when(c == _CNT_CORE[rel])
            def _(rel=rel, per_s=per_s, ngrp=ngrp):
                acc = accs[rel]
                base0 = s * per_s
                pltpu.async_copy(dd[rel].at[pl.ds(base0, KCT * 128)],
                                 dst2[0], pdsem[0])

                def grp(g, carry, rel=rel, base0=base0, ngrp=ngrp, acc=acc):
                    for buf in (0, 1):
                        @pl.when(g % 2 == buf)
                        def _(buf=buf):
                            dib = dst2[buf]
                            dio = dst2[1 - buf]
                            b0 = base0 + g * (KCT * 128)
                            pltpu.make_async_copy(
                                dd[rel].at[pl.ds(b0, KCT * 128)], dib,
                                pdsem[buf]).wait()
                            for j in range(KCT):
                                @pl.when(g > 0)
                                def _(j=j):
                                    pltpu.make_async_copy(
                                        onesb,
                                        acc.at[dio.at[pl.ds(j * 128, 128)]],
                                        ssem[j]).wait()

                            @pl.when(g + 1 < ngrp)
                            def _():
                                pltpu.async_copy(
                                    dd[rel].at[pl.ds(b0 + KCT * 128,
                                                     KCT * 128)],
                                    dio, pdsem[1 - buf])
                            for j in range(KCT):
                                pltpu.async_copy(
                                    onesb,
                                    acc.at[dib.at[pl.ds(j * 128, 128)]],
                                    ssem[j], add=True)
                    return carry

                lax.fori_loop(0, ngrp, grp, 0)
                dlast = dst2[(ngrp - 1) % 2]
                for j in range(KCT):
                    pltpu.make_async_copy(
                        onesb, acc.at[dlast.at[pl.ds(j * 128, 128)]],
                        ssem[j]).wait()

        plsc.subcore_barrier()
        for rel, (ep, npd, _) in enumerate(_RELS):
            stripe = npd // NSUB

            @pl.when(c == _CNT_CORE[rel])
            def _(rel=rel, stripe=stripe):
                acc = accs[rel]
                pltpu.sync_copy(acc.at[pl.ds(s * stripe, stripe)],
                                cbuf.at[pl.ds(0, stripe)])

                def rec(i, carry):
                    v = cbuf[pl.ds(i * 16, 16)]
                    cbuf[pl.ds(i * 16, 16)] = 1.0 / jnp.maximum(v, 1.0)
                    return carry

                lax.fori_loop(0, stripe // 16, rec, 0)
                pltpu.sync_copy(cbuf.at[pl.ds(0, stripe)],
                                outs[rel].at[pl.ds(s * stripe, stripe)])

    return k(dsts[0], dsts[1], dsts[2], dsts[3], zeros1d, ones128)


def _sc_decoder_gather(tables, idxs):
    out_type = [jax.ShapeDtypeStruct((EP_L, 128), F32) for _ in range(4)]
    scratch = ([pltpu.VMEM((KDE * 128,), I32) for _ in range(2)]
               + [pltpu.VMEM((128, 128), F32) for _ in range(KDE)]
               + [pltpu.SemaphoreType.DMA for _ in range(2 * KDE + 2)])

    @functools.partial(pl.kernel, out_type=out_type, mesh=_MESH,
                       scratch_types=scratch, compiler_params=_SC_PARAMS)
    def k(t0, t1, t2, t3, i0, i1, i2, i3, o0, o1, o2, o3, *rest):
        idx2 = rest[:2]
        rowb = rest[2:2 + KDE]
        gsem = rest[2 + KDE:2 + 2 * KDE]
        osem = rest[2 + 2 * KDE:2 + 3 * KDE]
        pisem = rest[2 + 3 * KDE:4 + 3 * KDE]
        c = lax.axis_index("c")
        s = lax.axis_index("s")
        w = s * 2 + c
        per_w = EP_L // 32
        ngrp = per_w // (KDE * 128)
        tabs = (t0, t1, t2, t3)
        iis = (i0, i1, i2, i3)
        outs = (o0, o1, o2, o3)
        for job in range(4):
            base0 = w * per_w
            pltpu.async_copy(iis[job].at[pl.ds(base0, KDE * 128)],
                             idx2[0], pisem[0])

            def grp(g, carry, job=job, base0=base0):
                for buf in (0, 1):
                    @pl.when(g % 2 == buf)
                    def _(buf=buf):
                        iib = idx2[buf]
                        iio = idx2[1 - buf]
                        b0 = base0 + g * (KDE * 128)
                        pltpu.make_async_copy(
                            iis[job].at[pl.ds(b0, KDE * 128)], iib,
                            pisem[buf]).wait()
                        for j in range(KDE):
                            @pl.when(g > 0)
                            def _(j=j):
                                pltpu.make_async_copy(
                                    rowb[j],
                                    outs[job].at[pl.ds(b0 - KDE * 128
                                                       + j * 128, 128)],
                                    osem[j]).wait()
                            pltpu.async_copy(
                                tabs[job].at[iib.at[pl.ds(j * 128, 128)]],
                                rowb[j], gsem[j])

                        @pl.when(g + 1 < ngrp)
                        def _():
                            pltpu.async_copy(
                                iis[job].at[pl.ds(b0 + KDE * 128, KDE * 128)],
                                iio, pisem[1 - buf])
                        for j in range(KDE):
                            pltpu.make_async_copy(
                                tabs[job].at[iib.at[pl.ds(j * 128, 128)]],
                                rowb[j], gsem[j]).wait()
                            pltpu.async_copy(
                                rowb[j],
                                outs[job].at[pl.ds(b0 + j * 128, 128)],
                                osem[j])
                return carry

            lax.fori_loop(0, ngrp, grp, 0)
            blast = base0 + (ngrp - 1) * (KDE * 128)
            for j in range(KDE):
                pltpu.make_async_copy(
                    rowb[j], outs[job].at[pl.ds(blast + j * 128, 128)],
                    osem[j]).wait()

    return k(tables[0], tables[1], tables[2], tables[3],
             idxs[0], idxs[1], idxs[2], idxs[3])


def _prep_edges(ei, n_src, n_dst, n_dst_pad, e_pad):
    src, dst = ei[0], ei[1]
    npad = e_pad - src.shape[0]
    ar = jnp.arange(npad, dtype=I32)
    srcp = jnp.concatenate([src, ar % jnp.int32(n_src)])
    dstp = jnp.concatenate([dst, n_dst + (ar % jnp.int32(n_dst_pad - n_dst))])
    gidx = srcp[None, :] * 4 + jnp.arange(4, dtype=I32)[:, None]
    return gidx, dstp


def _pad_idx(idx, n_rows, e_pad):
    npad = e_pad - idx.shape[0]
    ar = jnp.arange(npad, dtype=I32)
    return jnp.concatenate([idx, ar % jnp.int32(n_rows)])


def kernel(x_news, x_keyword, x_stock, ei_hk, ei_rhk, ei_hs, ei_rhs, eli_hk, eli_hs, news_W, news_b, key_W, key_b, stock_W, stock_b, c1_hk_Wl, c1_hk_bl, c1_hk_Wr, c1_rhk_Wl, c1_rhk_bl, c1_rhk_Wr, c1_hs_Wl, c1_hs_bl, c1_hs_Wr, c1_rhs_Wl, c1_rhs_bl, c1_rhs_Wr, c2_hk_Wl, c2_hk_bl, c2_hk_Wr, c2_rhk_Wl, c2_rhk_bl, c2_rhk_Wr, c2_hs_Wl, c2_hs_bl, c2_hs_Wr, c2_rhs_Wl, c2_rhs_bl, c2_rhs_Wr, dec_hk_W1, dec_hk_b1, dec_hk_W2, dec_hk_b2, dec_hs_W1, dec_hs_b1, dec_hs_W2, dec_hs_b2):
    h_news = tc_linear(x_news, news_W, news_b, act=True)
    h_key = tc_linear(x_keyword, key_W, key_b, act=True)
    h_stock = tc_linear(x_stock, stock_W, stock_b, act=True)

    zeros2d = jnp.zeros((512, 32), F32)
    zeros1d = jnp.zeros((4096,), F32)
    ones128 = jnp.ones((128,), F32)

    gd = [
        _prep_edges(ei_hk, N_NEWS, N_KEY, NP_KEY, EP_HK),
        _prep_edges(ei_rhk, N_KEY, N_NEWS, NP_NEWS, EP_HK),
        _prep_edges(ei_hs, N_NEWS, N_STOCK, NP_STOCK, EP_HS),
        _prep_edges(ei_rhs, N_STOCK, N_NEWS, NP_NEWS, EP_HS),
    ]
    ics = _sc_inv_counts([d for _, d in gd], zeros1d, ones128)
    ic_hk, ic_rhk, ic_hs, ic_rhs = [ic.reshape(-1, 1) for ic in ics]

    def layer(hn, hk, hs, Wls, bls, Wrs, act, posts=(None, None, None)):
        (wl_hk, wl_rhk, wl_hs, wl_rhs) = Wls
        (b_hk, b_rhk, b_hs, b_rhs) = bls
        (r_hk, r_rhk, r_hs, r_rhs) = Wrs
        ss = _sc_segsum_layer(hn.reshape(-1, 32), hk.reshape(-1, 32),
                              hs.reshape(-1, 32), gd, zeros2d)
        ss_hk, ss_rhk, ss_hs, ss_rhs = ss
        k = tc_sage_out([(ss_hk, ic_hk, wl_hk)], hk, r_hk, b_hk, act,
                        post=posts[1])
        n = tc_sage_out([(ss_rhk, ic_rhk, wl_rhk), (ss_rhs, ic_rhs, wl_rhs)],
                        hn, r_rhk + r_rhs, b_rhk + b_rhs, act, post=posts[0])
        s = tc_sage_out([(ss_hs, ic_hs, wl_hs)], hs, r_hs, b_hs, act,
                        post=posts[2])
        return n, k, s

    n1, k1, s1 = layer(
        h_news, h_key, h_stock,
        (c1_hk_Wl, c1_rhk_Wl, c1_hs_Wl, c1_rhs_Wl),
        (c1_hk_bl, c1_rhk_bl, c1_hs_bl, c1_rhs_bl),
        (c1_hk_Wr, c1_rhk_Wr, c1_hs_Wr, c1_rhs_Wr), True)
    post_n = (jnp.concatenate([dec_hk_W1[:, :128], dec_hs_W1[:, :128]], 0),
              jnp.zeros((256,), F32))
    post_k = (dec_hk_W1[:, 128:], dec_hk_b1)
    post_s = (dec_hs_W1[:, 128:], dec_hs_b1)
    za, b_hk, b_hs = layer(
        n1, k1, s1,
        (c2_hk_Wl, c2_rhk_Wl, c2_hs_Wl, c2_rhs_Wl),
        (c2_hk_bl, c2_rhk_bl, c2_hs_bl, c2_rhs_bl),
        (c2_hk_Wr, c2_rhk_Wr, c2_hs_Wr, c2_rhs_Wr), False,
        posts=(post_n, post_k, post_s))
    a2 = za.reshape(-1, 128)

    idxs = [2 * _pad_idx(eli_hk[0], N_NEWS, EP_L),
            _pad_idx(eli_hk[1], N_KEY, EP_L),
            2 * _pad_idx(eli_hs[0], N_NEWS, EP_L) + 1,
            _pad_idx(eli_hs[1], N_STOCK, EP_L)]
    ga_hk, gb_hk, ga_hs, gb_hs = _sc_decoder_gather(
        (a2, b_hk, a2, b_hs), idxs)

    pred_hk, pred_hs = tc_decode2(
        ga_hk, gb_hk, ga_hs, gb_hs,
        jnp.stack([dec_hk_W2[0], dec_hs_W2[0]]),
        jnp.concatenate([dec_hk_b2, dec_hs_b2]))
    return (pred_hk[:E_L], pred_hs[:E_L])
